# Initial kernel scaffold; baseline (speedup 1.0000x reference)
import jax
import jax.numpy as jnp
from jax.experimental import pallas as pl


def kernel(x, router_logits, gate_w, up_w, down_w):
    raise NotImplementedError("write your pallas kernel here")


# speedup vs baseline: 1.9008x; 1.9008x over previous
import functools

import jax
import jax.numpy as jnp
from jax.experimental import pallas as pl
from jax.experimental.pallas import tpu as pltpu

E = 16
H = 1024
F = 2816
T = 2048
F_BLOCKS = 11
FB = F // F_BLOCKS


def _moe_body(router_ref, x_ref, gw_ref, uw_ref, dw_ref, out_ref):
    e = pl.program_id(0)
    f = pl.program_id(1)

    @pl.when(jnp.logical_and(e == 0, f == 0))
    def _init():
        out_ref[...] = jnp.zeros_like(out_ref)

    top = jnp.argmax(router_ref[...], axis=-1)
    mask = (top == e).astype(jnp.float32)[:, None]
    xm = x_ref[...] * mask
    g = jax.lax.dot_general(
        xm, gw_ref[0], (((1,), (1,)), ((), ())),
        preferred_element_type=jnp.float32)
    u = jax.lax.dot_general(
        xm, uw_ref[0], (((1,), (1,)), ((), ())),
        preferred_element_type=jnp.float32)
    h = (g * jax.nn.sigmoid(g)) * u
    out_ref[...] += jax.lax.dot_general(
        h, dw_ref[0], (((1,), (1,)), ((), ())),
        preferred_element_type=jnp.float32)


@jax.jit
def kernel(x, router_logits, gate_w, up_w, down_w):
    grid = (E, F_BLOCKS)
    return pl.pallas_call(
        _moe_body,
        grid=grid,
        in_specs=[
            pl.BlockSpec((T, E), lambda e, f: (0, 0)),
            pl.BlockSpec((T, H), lambda e, f: (0, 0)),
            pl.BlockSpec((1, FB, H), lambda e, f: (e, f, 0)),
            pl.BlockSpec((1, FB, H), lambda e, f: (e, f, 0)),
            pl.BlockSpec((1, H, FB), lambda e, f: (e, 0, f)),
        ],
        out_specs=pl.BlockSpec((T, H), lambda e, f: (0, 0)),
        out_shape=jax.ShapeDtypeStruct((T, H), jnp.float32),
    )(router_logits, x, gate_w, up_w, down_w)


# speedup vs baseline: 4.2149x; 2.2174x over previous
import functools

import jax
import jax.numpy as jnp
from jax import lax
from jax.experimental import pallas as pl
from jax.experimental.pallas import tpu as pltpu
from jax.experimental.pallas import tpu_sc as plsc

E = 16
H = 1024
F = 2816
T = 2048
RT = 256
NB = T // RT
NI = NB + E - 1
FH = F // 2
NTILES = 16
TPT = T // NTILES
NCHUNK = TPT // 16


def _excl_cumsum16(x):
    iota = lax.iota(jnp.int32, 16)
    for d in (1, 2, 4, 8):
        x = x + jnp.where(iota >= d, x[jnp.maximum(iota - d, 0)],
                          jnp.zeros_like(x))
    return jnp.where(iota >= 1, x[jnp.maximum(iota - 1, 0)],
                     jnp.zeros_like(x))


def _vec(s):
    return jnp.zeros(16, jnp.int32) + s


def _sc_count_body(router_hbm, cnt_hbm, expd_hbm, rankd_hbm,
                   router_v, expv, rankv, cnt_vm, sem):
    c = lax.axis_index("c")
    s = lax.axis_index("s")

    @pl.when(c == 0)
    def _core0():
        iota = lax.iota(jnp.int32, 16)
        base_tok = s * TPT
        pltpu.sync_copy(router_hbm.at[s], router_v)

        for ch in range(NCHUNK):
            best = router_v[0, pl.ds(ch * 16, 16)]
            bidx = jnp.zeros(16, jnp.int32)
            for e in range(1, E):
                col = router_v[e, pl.ds(ch * 16, 16)]
                upd = col > best
                best = jnp.where(upd, col, best)
                bidx = jnp.where(upd, e, bidx)
            expv[pl.ds(ch * 16, 16)] = bidx

        running = jnp.zeros(16, jnp.int32)
        for ch in range(NCHUNK):
            e_ch = expv[pl.ds(ch * 16, 16)]
            rank_ch = jnp.zeros(16, jnp.int32)
            for lane in range(16):
                e_splat = e_ch[jnp.full((16,), lane, jnp.int32)]
                rank_ch = jnp.where(iota == lane, running[e_splat], rank_ch)
                running = running + jnp.where(iota == e_splat, 1, 0)
            rankv[pl.ds(ch * 16, 16)] = rank_ch

        cnt_vm[...] = running
        pltpu.sync_copy(cnt_vm, cnt_hbm.at[pl.ds(s * 16, 16)])
        pltpu.sync_copy(expv, expd_hbm.at[pl.ds(base_tok, TPT)])
        pltpu.sync_copy(rankv, rankd_hbm.at[pl.ds(base_tok, TPT)])


@functools.partial(
    pl.kernel,
    out_type=[
        jax.ShapeDtypeStruct((NTILES * 16,), jnp.int32),
        jax.ShapeDtypeStruct((T,), jnp.int32),
        jax.ShapeDtypeStruct((T,), jnp.int32),
    ],
    mesh=plsc.VectorSubcoreMesh(core_axis_name="c", subcore_axis_name="s"),
    scratch_types=[
        pltpu.VMEM((E, TPT), jnp.float32),
        pltpu.VMEM((TPT,), jnp.int32),
        pltpu.VMEM((TPT,), jnp.int32),
        pltpu.VMEM((16,), jnp.int32),
        pltpu.SemaphoreType.DMA,
    ],
)
def _sc_count(router_hbm, cnt_hbm, expd_hbm, rankd_hbm, *scratch):
    _sc_count_body(router_hbm, cnt_hbm, expd_hbm, rankd_hbm, *scratch)


def _sc_pos_body(cnt_hbm, expd_hbm, rankd_hbm, pos_hbm, off_hbm,
                 expv, rankv, posv, cnt_vm, all_cnt, sem):
    c = lax.axis_index("c")
    s = lax.axis_index("s")

    @pl.when(c == 0)
    def _core0():
        base_tok = s * TPT
        pltpu.sync_copy(cnt_hbm, all_cnt)
        pltpu.sync_copy(expd_hbm.at[pl.ds(base_tok, TPT)], expv)
        pltpu.sync_copy(rankd_hbm.at[pl.ds(base_tok, TPT)], rankv)
        totals = jnp.zeros(16, jnp.int32)
        prefix = jnp.zeros(16, jnp.int32)
        for w in range(NTILES):
            row = all_cnt[pl.ds(w * 16, 16)]
            totals = totals + row
            prefix = prefix + row * _vec(jnp.where(w < s, 1, 0))
        off = _excl_cumsum16(totals)
        tile_base = off + prefix

        for ch in range(NCHUNK):
            e_ch = expv[pl.ds(ch * 16, 16)]
            posv[pl.ds(ch * 16, 16)] = (
                tile_base[e_ch] + rankv[pl.ds(ch * 16, 16)])
        pltpu.sync_copy(posv, pos_hbm.at[pl.ds(base_tok, TPT)])

        @pl.when(s == 0)
        def _offs():
            cnt_vm[...] = off
            pltpu.sync_copy(cnt_vm, off_hbm)


@functools.partial(
    pl.kernel,
    out_type=[
        jax.ShapeDtypeStruct((T,), jnp.int32),
        jax.ShapeDtypeStruct((16,), jnp.int32),
    ],
    mesh=plsc.VectorSubcoreMesh(core_axis_name="c", subcore_axis_name="s"),
    scratch_types=[
        pltpu.VMEM((TPT,), jnp.int32),
        pltpu.VMEM((TPT,), jnp.int32),
        pltpu.VMEM((TPT,), jnp.int32),
        pltpu.VMEM((16,), jnp.int32),
        pltpu.VMEM((NTILES * 16,), jnp.int32),
        pltpu.SemaphoreType.DMA,
    ],
)
def _sc_pos(cnt_hbm, expd_hbm, rankd_hbm, pos_hbm, off_hbm, *scratch):
    _sc_pos_body(cnt_hbm, expd_hbm, rankd_hbm, pos_hbm, off_hbm, *scratch)


def _tc_meta_body(off_ref, out_ref):
    col = lax.broadcasted_iota(jnp.int32, (4, 32), 1)
    row = lax.broadcasted_iota(jnp.int32, (4, 32), 0)

    def off_at(e):
        return jnp.where(e >= E, T, off_ref[jnp.minimum(e, E - 1)])

    def step(i, st):
        e, blk, acc = st
        oe = off_at(e)
        oe1 = off_at(e + 1)
        active = e < E
        lo = jnp.where(active, jnp.maximum(oe, blk * RT), 0)
        hi = jnp.where(active, jnp.minimum(oe1, (blk + 1) * RT), 0)
        e_emit = jnp.where(active, e, E - 1)
        blk_emit = jnp.where(active, blk, NB - 1)
        vals = jnp.where(row == 0, e_emit,
                         jnp.where(row == 1, blk_emit,
                                   jnp.where(row == 2, lo, hi)))
        acc = jnp.where(col == i, vals, acc)
        done = hi >= oe1
        e_n = jnp.where(jnp.logical_and(active, done), e + 1, e)
        blk_n = jnp.where(done, jnp.minimum(oe1 // RT, NB - 1), blk + 1)
        e_n = jnp.where(active, e_n, e)
        blk_n = jnp.where(active, blk_n, blk)
        return (e_n, blk_n, acc)

    init = jnp.where(row == 0, E - 1,
                     jnp.where(row == 1, NB - 1, 0)).astype(jnp.int32)
    _, _, acc = lax.fori_loop(0, NI, step, (jnp.int32(0), jnp.int32(0), init))
    out_ref[...] = acc


def _item_meta(off):
    grid_spec = pltpu.PrefetchScalarGridSpec(
        num_scalar_prefetch=1,
        grid=(1,),
        in_specs=[],
        out_specs=pl.BlockSpec((4, 32), lambda i, o: (0, 0)),
    )
    return pl.pallas_call(
        _tc_meta_body,
        grid_spec=grid_spec,
        out_shape=jax.ShapeDtypeStruct((4, 32), jnp.int32),
    )(off)


def _tc_group_body(item_e, item_blk, item_lo, item_hi,
                   pos_ref, x_ref, gw_ref, uw_ref, dw_ref, out_ref, xm_ref):
    i = pl.program_id(0)
    f = pl.program_id(1)
    blk = item_blk[i]
    lo = item_lo[i]
    hi = item_hi[i]
    rows = blk * RT + lax.broadcasted_iota(jnp.int32, (RT, 1), 0)

    @pl.when(f == 0)
    def _gather_rows():
        onehot = jnp.where(
            jnp.logical_and(pos_ref[...] == rows,
                            jnp.logical_and(rows >= lo, rows < hi)),
            1.0, 0.0)
        xm_ref[...] = lax.dot_general(
            onehot, x_ref[...], (((1,), (0,)), ((), ())),
            preferred_element_type=jnp.float32)

    xm = xm_ref[...]
    g = lax.dot_general(xm, gw_ref[0], (((1,), (1,)), ((), ())),
                        preferred_element_type=jnp.float32)
    u = lax.dot_general(xm, uw_ref[0], (((1,), (1,)), ((), ())),
                        preferred_element_type=jnp.float32)
    h = g * jax.nn.sigmoid(g) * u
    y = lax.dot_general(h, dw_ref[0], (((1,), (1,)), ((), ())),
                        preferred_element_type=jnp.float32)
    first = jnp.logical_and(
        f == 0,
        jnp.logical_or(i == 0,
                       item_blk[i] != item_blk[jnp.maximum(i - 1, 0)]))

    @pl.when(first)
    def _init():
        out_ref[...] = y

    @pl.when(jnp.logical_not(first))
    def _acc():
        out_ref[...] += y


def _grouped_mlp(item_e, item_blk, item_lo, item_hi, pos2, x,
                 gate_w, up_w, down_w):
    grid_spec = pltpu.PrefetchScalarGridSpec(
        num_scalar_prefetch=4,
        grid=(NI, 2),
        in_specs=[
            pl.BlockSpec((1, T), lambda i, f, ie, ib, il, ih: (0, 0)),
            pl.BlockSpec((T, H), lambda i, f, ie, ib, il, ih: (0, 0)),
            pl.BlockSpec((1, FH, H),
                         lambda i, f, ie, ib, il, ih: (ie[i], f, 0)),
            pl.BlockSpec((1, FH, H),
                         lambda i, f, ie, ib, il, ih: (ie[i], f, 0)),
            pl.BlockSpec((1, H, FH),
                         lambda i, f, ie, ib, il, ih: (ie[i], 0, f)),
        ],
        out_specs=pl.BlockSpec((RT, H),
                               lambda i, f, ie, ib, il, ih: (ib[i], 0)),
        scratch_shapes=[pltpu.VMEM((RT, H), jnp.float32)],
    )
    return pl.pallas_call(
        _tc_group_body,
        grid_spec=grid_spec,
        out_shape=jax.ShapeDtypeStruct((T, H), jnp.float32),
    )(item_e, item_blk, item_lo, item_hi, pos2, x, gate_w, up_w, down_w)


@functools.partial(
    pl.kernel,
    out_type=jax.ShapeDtypeStruct((T, H), jnp.float32),
    mesh=plsc.VectorSubcoreMesh(core_axis_name="c", subcore_axis_name="s"),
    scratch_types=[
        pltpu.VMEM((64,), jnp.int32),
        pltpu.VMEM((64, H), jnp.float32),
        pltpu.SemaphoreType.DMA,
    ],
)
def _unsort(pos_hbm, ys_hbm, out_hbm, posbuf, rows_v, sem):
    c = lax.axis_index("c")
    s = lax.axis_index("s")
    w = s * 2 + c
    base = w * 64
    pltpu.sync_copy(pos_hbm.at[pl.ds(base, 64)], posbuf)
    pltpu.async_copy(ys_hbm.at[posbuf], rows_v, sem).wait()
    pltpu.sync_copy(rows_v, out_hbm.at[pl.ds(base, 64)])


@jax.jit
def kernel(x, router_logits, gate_w, up_w, down_w):
    rt = router_logits.T.reshape(E, NTILES, TPT).transpose(1, 0, 2)
    cnt, expd, rankd = _sc_count(rt)
    pos, off = _sc_pos(cnt, expd, rankd)
    meta = _item_meta(off)
    ys = _grouped_mlp(meta[0], meta[1], meta[2], meta[3],
                      pos.reshape(1, T), x, gate_w, up_w, down_w)
    return _unsort(pos, ys)
